# Initial kernel scaffold; baseline (speedup 1.0000x reference)
import jax
import jax.numpy as jnp
from jax.experimental import pallas as pl


def kernel(x, edge_index, W1, b1, W2, b2, W3, b3):
    raise NotImplementedError("write your pallas kernel here")


# speedup vs baseline: 12.9908x; 12.9908x over previous
import functools

import jax
import jax.numpy as jnp
from jax import lax
from jax.experimental import pallas as pl
from jax.experimental.pallas import tpu as pltpu
from jax.experimental.pallas import tpu_sc as plsc

N = 10000
E = 160000
NC, NS, LANES = 2, 16, 16
NW = NC * NS
EPT = E // NW
BATCH = 128
NB = 40
EPAD = NB * BATCH
PADV = 16
NPAD = 10240
SPAN = NPAD // NS
BM = 400

_MESH = plsc.VectorSubcoreMesh(
    core_axis_name="c", subcore_axis_name="s", num_cores=NC, num_subcores=NS
)


def _zero_fill(buf, rows, cw):
    z16 = jnp.zeros((LANES,), jnp.float32)

    def body(r, _):
        for k in range(cw // LANES):
            buf[r, pl.ds(k * LANES, LANES)] = z16
        return 0

    lax.fori_loop(0, rows, body, 0)


CW = 128


def _make_propagate(C):
    @functools.partial(
        pl.kernel,
        out_type=jax.ShapeDtypeStruct((NC, C, NPAD, CW), jnp.float32),
        mesh=_MESH,
        scratch_types=[
            pltpu.VMEM((NB, BATCH), jnp.int32),
            pltpu.VMEM((NB, BATCH), jnp.int32),
            pltpu.VMEM((BATCH, CW), jnp.float32),
            pltpu.VMEM((BATCH, CW), jnp.float32),
            pltpu.VMEM_SHARED((NPAD, CW), jnp.float32),
            pltpu.SemaphoreType.DMA,
            pltpu.SemaphoreType.DMA,
        ],
    )
    def prop(table, src_p, dst_p, out, sidx_v, didx_v, rows0, rows1,
             acc, gsem0, gsem1):
        core = lax.axis_index("c")
        sid = lax.axis_index("s")
        pltpu.sync_copy(dst_p.at[core, sid], didx_v)
        pltpu.sync_copy(src_p.at[core, sid], sidx_v)
        if C > 1:
            def mulbody(r, _):
                for k in range(BATCH // LANES):
                    s = sidx_v[r, pl.ds(k * LANES, LANES)]
                    sidx_v[r, pl.ds(k * LANES, LANES)] = s * C
                return 0

            lax.fori_loop(0, NB, mulbody, 0)
        rows = (rows0, rows1)
        gsems = (gsem0, gsem1)
        for c in range(C):
            _zero_fill(rows0, BATCH, CW)
            for z in range(SPAN // BATCH):
                pltpu.sync_copy(
                    rows0, acc.at[pl.ds(sid * SPAN + z * BATCH, BATCH)])
            if c > 0:
                def incbody(r, _):
                    for k in range(BATCH // LANES):
                        s = sidx_v[r, pl.ds(k * LANES, LANES)]
                        sidx_v[r, pl.ds(k * LANES, LANES)] = s + 1
                    return 0

                lax.fori_loop(0, NB, incbody, 0)
            plsc.subcore_barrier()
            cps = [None, None]
            cps[0] = pltpu.async_copy(table.at[sidx_v.at[0]], rows[0],
                                      gsems[0])
            for j in range(NB):
                b = j % 2
                cps[b].wait()
                if j + 1 < NB:
                    cps[1 - b] = pltpu.async_copy(
                        table.at[sidx_v.at[j + 1]], rows[1 - b], gsems[1 - b])
                pltpu.sync_copy(rows[b], acc.at[didx_v.at[j]], add=True)
            plsc.subcore_barrier()
            pltpu.sync_copy(acc.at[pl.ds(sid * SPAN, SPAN)],
                            out.at[core, c, pl.ds(sid * SPAN, SPAN)])

    return prop


C1 = 2
C2 = 4
_prop_l1 = _make_propagate(C1)
_prop_l2 = _make_propagate(C2)
_prop_l3 = _make_propagate(1)


@functools.partial(
    pl.kernel,
    out_type=jax.ShapeDtypeStruct((NC, NPAD), jnp.float32),
    mesh=_MESH,
    scratch_types=[
        pltpu.VMEM((NB, BATCH), jnp.int32),
        pltpu.VMEM((BATCH,), jnp.float32),
        pltpu.VMEM((SPAN,), jnp.float32),
        pltpu.VMEM_SHARED((NPAD,), jnp.float32),
    ],
)
def _deg_kernel(dst_idx, out, didx_v, ones_v, zer_v, dacc):
    core = lax.axis_index("c")
    sid = lax.axis_index("s")
    one16 = jnp.ones((LANES,), jnp.float32)
    z16 = jnp.zeros((LANES,), jnp.float32)
    for k in range(BATCH // LANES):
        ones_v[pl.ds(k * LANES, LANES)] = one16
    def zbody(i, _):
        zer_v[pl.ds(i * LANES, LANES)] = z16
        return 0

    lax.fori_loop(0, SPAN // LANES, zbody, 0)
    pltpu.sync_copy(zer_v, dacc.at[pl.ds(sid * SPAN, SPAN)])
    plsc.subcore_barrier()
    pltpu.sync_copy(dst_idx.at[core, sid], didx_v)
    for j in range(NB):
        pltpu.sync_copy(ones_v, dacc.at[didx_v.at[j]], add=True)
    plsc.subcore_barrier()
    pltpu.sync_copy(dacc.at[pl.ds(sid * SPAN, SPAN)],
                    out.at[core, pl.ds(sid * SPAN, SPAN)])


def _k0_body(d0, d1, x, dinv_o, g1_o):
    deg = d0[0] + d1[0] + 1.0
    dv = lax.rsqrt(deg)
    dinv_o[...] = dv
    g1_o[...] = x[...] * dv


def _k0(degp, x):
    return pl.pallas_call(
        _k0_body,
        grid=(N // BM,),
        in_specs=[
            pl.BlockSpec((1, BM, 1), lambda i: (0, i, 0)),
            pl.BlockSpec((1, BM, 1), lambda i: (1, i, 0)),
            pl.BlockSpec((BM, 256), lambda i: (i, 0)),
        ],
        out_specs=[
            pl.BlockSpec((BM, 1), lambda i: (i, 0)),
            pl.BlockSpec((BM, 256), lambda i: (i, 0)),
        ],
        out_shape=[
            jax.ShapeDtypeStruct((N, 1), jnp.float32),
            jax.ShapeDtypeStruct((N, 256), jnp.float32),
        ],
    )(degp, degp, x)


def _part_specs(C):
    specs = []
    for c in range(C):
        for core in range(NC):
            specs.append(pl.BlockSpec(
                (1, 1, BM, CW),
                functools.partial(lambda i, _co=0, _ch=0: (_co, _ch, i, 0),
                                  _co=core, _ch=c)))
    return specs


def _psum(ps, C):
    return jnp.concatenate(
        [ps[2 * c][0, 0] + ps[2 * c + 1][0, 0] for c in range(C)], axis=1)


def _k1_body(*refs):
    ps = refs[:2 * C1]
    g1, dinv, w1, b1, g2_o = refs[2 * C1:]
    pre = _psum(ps, C1) + g1[...]
    dv = dinv[...]
    pre = pre * dv
    h = jnp.dot(pre, w1[...], preferred_element_type=jnp.float32) + b1[...]
    g2_o[...] = jnp.maximum(h, 0.0) * dv


def _k1(parts1, g1, dinv, W1, b1):
    return pl.pallas_call(
        _k1_body,
        grid=(N // BM,),
        in_specs=_part_specs(C1) + [
            pl.BlockSpec((BM, 256), lambda i: (i, 0)),
            pl.BlockSpec((BM, 1), lambda i: (i, 0)),
            pl.BlockSpec((256, 512), lambda i: (0, 0)),
            pl.BlockSpec((1, 512), lambda i: (0, 0)),
        ],
        out_specs=pl.BlockSpec((BM, 512), lambda i: (i, 0)),
        out_shape=jax.ShapeDtypeStruct((N, 512), jnp.float32),
    )(*([parts1] * (2 * C1)), g1, dinv, W1, b1)


def _k2_body(*refs):
    ps = refs[:2 * C2]
    g2, dinv, w2, b2, w3, g3_o = refs[2 * C2:]
    pre = _psum(ps, C2) + g2[...]
    dv = dinv[...]
    pre = pre * dv
    h = jnp.maximum(
        jnp.dot(pre, w2[...], preferred_element_type=jnp.float32) + b2[...],
        0.0)
    g3_o[...] = jnp.dot(h, w3[...], preferred_element_type=jnp.float32) * dv


def _k2(parts2, g2, dinv, W2, b2, W3p):
    return pl.pallas_call(
        _k2_body,
        grid=(N // BM,),
        in_specs=_part_specs(C2) + [
            pl.BlockSpec((BM, 512), lambda i: (i, 0)),
            pl.BlockSpec((BM, 1), lambda i: (i, 0)),
            pl.BlockSpec((512, 512), lambda i: (0, 0)),
            pl.BlockSpec((1, 512), lambda i: (0, 0)),
            pl.BlockSpec((512, CW), lambda i: (0, 0)),
        ],
        out_specs=pl.BlockSpec((BM, CW), lambda i: (i, 0)),
        out_shape=jax.ShapeDtypeStruct((N, CW), jnp.float32),
    )(*([parts2] * (2 * C2)), g2, dinv, W2, b2, W3p)


def _k3_body(p0, p1, g3, dinv, b3, out_o):
    out_o[...] = (p0[0, 0] + p1[0, 0] + g3[...]) * dinv[...] + b3[...]


def _k3(parts3, g3, dinv, b3p):
    return pl.pallas_call(
        _k3_body,
        grid=(N // BM,),
        in_specs=[
            pl.BlockSpec((1, 1, BM, CW), lambda i: (0, 0, i, 0)),
            pl.BlockSpec((1, 1, BM, CW), lambda i: (1, 0, i, 0)),
            pl.BlockSpec((BM, CW), lambda i: (i, 0)),
            pl.BlockSpec((BM, 1), lambda i: (i, 0)),
            pl.BlockSpec((1, CW), lambda i: (0, 0)),
        ],
        out_specs=pl.BlockSpec((BM, CW), lambda i: (i, 0)),
        out_shape=jax.ShapeDtypeStruct((N, CW), jnp.float32),
    )(parts3, parts3, g3, dinv, b3p)


def _pad_tiles(a, padvals):
    t = a.reshape(NW, EPT)
    pad = jnp.broadcast_to(padvals, (NW, EPAD - EPT)).astype(jnp.int32)
    return jnp.concatenate([t, pad], axis=1).reshape(NC, NS, NB, BATCH)


def _table(g, C):
    flat = g.reshape(N * C, CW)
    return jnp.concatenate([flat, jnp.zeros((PADV * C, CW), jnp.float32)],
                           axis=0)


def kernel(x, edge_index, W1, b1, W2, b2, W3, b3):
    src = edge_index[0].astype(jnp.int32)
    dst = edge_index[1].astype(jnp.int32)
    k_pad = jnp.arange(EPAD - EPT, dtype=jnp.int32)
    src_p = _pad_tiles(src, N + (k_pad % PADV))
    dst_p = _pad_tiles(dst, N + (k_pad % (NPAD - N)))

    degp = _deg_kernel(dst_p)
    dinv, g1 = _k0(degp.reshape(NC, NPAD, 1), x)

    parts1 = _prop_l1(_table(g1, C1), src_p, dst_p)
    g2 = _k1(parts1, g1, dinv, W1, b1.reshape(1, 512))

    parts2 = _prop_l2(_table(g2, C2), src_p, dst_p)
    W3p = jnp.concatenate([W3, jnp.zeros((512, CW - 2), jnp.float32)], axis=1)
    g3 = _k2(parts2, g2, dinv, W2, b2.reshape(1, 512), W3p)

    parts3 = _prop_l3(_table(g3, 1), src_p, dst_p)
    b3p = jnp.concatenate([b3, jnp.zeros((CW - 2,), jnp.float32)]).reshape(1, CW)
    out128 = _k3(parts3, g3, dinv, b3p)
    return out128[:, :2]


# speedup vs baseline: 13.0764x; 1.0066x over previous
import functools

import jax
import jax.numpy as jnp
from jax import lax
from jax.experimental import pallas as pl
from jax.experimental.pallas import tpu as pltpu
from jax.experimental.pallas import tpu_sc as plsc

N = 10000
E = 160000
NC, NS, LANES = 2, 16, 16
NW = NC * NS
EPT = E // NW
BATCH = 128
NB = 40
EPAD = NB * BATCH
PADV = 16
NPAD = 10240
SPAN = NPAD // NS
BM = 400

_MESH = plsc.VectorSubcoreMesh(
    core_axis_name="c", subcore_axis_name="s", num_cores=NC, num_subcores=NS
)


def _zero_fill(buf, rows, cw):
    z16 = jnp.zeros((LANES,), jnp.float32)

    def body(r, _):
        for k in range(cw // LANES):
            buf[r, pl.ds(k * LANES, LANES)] = z16
        return 0

    lax.fori_loop(0, rows, body, 0)


CW = 128


def _make_propagate(C):
    @functools.partial(
        pl.kernel,
        out_type=jax.ShapeDtypeStruct((NC, C, NPAD, CW), jnp.float32),
        mesh=_MESH,
        scratch_types=[
            pltpu.VMEM((NB, BATCH), jnp.int32),
            pltpu.VMEM((NB, BATCH), jnp.int32),
            pltpu.VMEM((BATCH, CW), jnp.float32),
            pltpu.VMEM((BATCH, CW), jnp.float32),
            pltpu.VMEM_SHARED((NPAD, CW), jnp.float32),
            pltpu.SemaphoreType.DMA,
            pltpu.SemaphoreType.DMA,
            pltpu.SemaphoreType.DMA,
            pltpu.SemaphoreType.DMA,
        ],
    )
    def prop(table, src_p, dst_p, out, sidx_v, didx_v, rows0, rows1,
             acc, gsem0, gsem1, ssem0, ssem1):
        core = lax.axis_index("c")
        sid = lax.axis_index("s")
        pltpu.sync_copy(dst_p.at[core, sid], didx_v)
        pltpu.sync_copy(src_p.at[core, sid], sidx_v)
        if C > 1:
            def mulbody(r, _):
                for k in range(BATCH // LANES):
                    s = sidx_v[r, pl.ds(k * LANES, LANES)]
                    sidx_v[r, pl.ds(k * LANES, LANES)] = s * C
                return 0

            lax.fori_loop(0, NB, mulbody, 0)
        rows = (rows0, rows1)
        gsems = (gsem0, gsem1)
        ssems = (ssem0, ssem1)
        for c in range(C):
            _zero_fill(rows0, BATCH, CW)
            for z in range(SPAN // BATCH):
                pltpu.sync_copy(
                    rows0, acc.at[pl.ds(sid * SPAN + z * BATCH, BATCH)])
            if c > 0:
                def incbody(r, _):
                    for k in range(BATCH // LANES):
                        s = sidx_v[r, pl.ds(k * LANES, LANES)]
                        sidx_v[r, pl.ds(k * LANES, LANES)] = s + 1
                    return 0

                lax.fori_loop(0, NB, incbody, 0)
            plsc.subcore_barrier()
            gcp = [None, None]
            scp = [None, None]
            gcp[0] = pltpu.async_copy(table.at[sidx_v.at[0]], rows[0],
                                      gsems[0])
            for j in range(NB):
                b = j % 2
                o = 1 - b
                gcp[b].wait()
                scp[b] = pltpu.async_copy(rows[b], acc.at[didx_v.at[j]],
                                          ssems[b], add=True)
                if j + 1 < NB:
                    if scp[o] is not None:
                        scp[o].wait()
                        scp[o] = None
                    gcp[o] = pltpu.async_copy(
                        table.at[sidx_v.at[j + 1]], rows[o], gsems[o])
            for b in range(2):
                if scp[b] is not None:
                    scp[b].wait()
            plsc.subcore_barrier()
            pltpu.sync_copy(acc.at[pl.ds(sid * SPAN, SPAN)],
                            out.at[core, c, pl.ds(sid * SPAN, SPAN)])

    return prop


C1 = 2
C2 = 4
_prop_l1 = _make_propagate(C1)
_prop_l2 = _make_propagate(C2)
_prop_l3 = _make_propagate(1)


@functools.partial(
    pl.kernel,
    out_type=jax.ShapeDtypeStruct((NC, NPAD), jnp.float32),
    mesh=_MESH,
    scratch_types=[
        pltpu.VMEM((NB, BATCH), jnp.int32),
        pltpu.VMEM((BATCH,), jnp.float32),
        pltpu.VMEM((SPAN,), jnp.float32),
        pltpu.VMEM_SHARED((NPAD,), jnp.float32),
    ],
)
def _deg_kernel(dst_idx, out, didx_v, ones_v, zer_v, dacc):
    core = lax.axis_index("c")
    sid = lax.axis_index("s")
    one16 = jnp.ones((LANES,), jnp.float32)
    z16 = jnp.zeros((LANES,), jnp.float32)
    for k in range(BATCH // LANES):
        ones_v[pl.ds(k * LANES, LANES)] = one16
    def zbody(i, _):
        zer_v[pl.ds(i * LANES, LANES)] = z16
        return 0

    lax.fori_loop(0, SPAN // LANES, zbody, 0)
    pltpu.sync_copy(zer_v, dacc.at[pl.ds(sid * SPAN, SPAN)])
    plsc.subcore_barrier()
    pltpu.sync_copy(dst_idx.at[core, sid], didx_v)
    for j in range(NB):
        pltpu.sync_copy(ones_v, dacc.at[didx_v.at[j]], add=True)
    plsc.subcore_barrier()
    pltpu.sync_copy(dacc.at[pl.ds(sid * SPAN, SPAN)],
                    out.at[core, pl.ds(sid * SPAN, SPAN)])


def _k0_body(d0, d1, x, dinv_o, g1_o):
    deg = d0[0] + d1[0] + 1.0
    dv = lax.rsqrt(deg)
    dinv_o[...] = dv
    g1_o[...] = x[...] * dv


def _k0(degp, x):
    return pl.pallas_call(
        _k0_body,
        grid=(N // BM,),
        in_specs=[
            pl.BlockSpec((1, BM, 1), lambda i: (0, i, 0)),
            pl.BlockSpec((1, BM, 1), lambda i: (1, i, 0)),
            pl.BlockSpec((BM, 256), lambda i: (i, 0)),
        ],
        out_specs=[
            pl.BlockSpec((BM, 1), lambda i: (i, 0)),
            pl.BlockSpec((BM, 256), lambda i: (i, 0)),
        ],
        out_shape=[
            jax.ShapeDtypeStruct((N, 1), jnp.float32),
            jax.ShapeDtypeStruct((N, 256), jnp.float32),
        ],
    )(degp, degp, x)


def _part_specs(C):
    specs = []
    for c in range(C):
        for core in range(NC):
            specs.append(pl.BlockSpec(
                (1, 1, BM, CW),
                functools.partial(lambda i, _co=0, _ch=0: (_co, _ch, i, 0),
                                  _co=core, _ch=c)))
    return specs


def _psum(ps, C):
    return jnp.concatenate(
        [ps[2 * c][0, 0] + ps[2 * c + 1][0, 0] for c in range(C)], axis=1)


def _k1_body(*refs):
    ps = refs[:2 * C1]
    g1, dinv, w1, b1, g2_o = refs[2 * C1:]
    pre = _psum(ps, C1) + g1[...]
    dv = dinv[...]
    pre = pre * dv
    h = jnp.dot(pre, w1[...], preferred_element_type=jnp.float32) + b1[...]
    g2_o[...] = jnp.maximum(h, 0.0) * dv


def _k1(parts1, g1, dinv, W1, b1):
    return pl.pallas_call(
        _k1_body,
        grid=(N // BM,),
        in_specs=_part_specs(C1) + [
            pl.BlockSpec((BM, 256), lambda i: (i, 0)),
            pl.BlockSpec((BM, 1), lambda i: (i, 0)),
            pl.BlockSpec((256, 512), lambda i: (0, 0)),
            pl.BlockSpec((1, 512), lambda i: (0, 0)),
        ],
        out_specs=pl.BlockSpec((BM, 512), lambda i: (i, 0)),
        out_shape=jax.ShapeDtypeStruct((N, 512), jnp.float32),
    )(*([parts1] * (2 * C1)), g1, dinv, W1, b1)


def _k2_body(*refs):
    ps = refs[:2 * C2]
    g2, dinv, w2, b2, w3, g3_o = refs[2 * C2:]
    pre = _psum(ps, C2) + g2[...]
    dv = dinv[...]
    pre = pre * dv
    h = jnp.maximum(
        jnp.dot(pre, w2[...], preferred_element_type=jnp.float32) + b2[...],
        0.0)
    g3_o[...] = jnp.dot(h, w3[...], preferred_element_type=jnp.float32) * dv


def _k2(parts2, g2, dinv, W2, b2, W3p):
    return pl.pallas_call(
        _k2_body,
        grid=(N // BM,),
        in_specs=_part_specs(C2) + [
            pl.BlockSpec((BM, 512), lambda i: (i, 0)),
            pl.BlockSpec((BM, 1), lambda i: (i, 0)),
            pl.BlockSpec((512, 512), lambda i: (0, 0)),
            pl.BlockSpec((1, 512), lambda i: (0, 0)),
            pl.BlockSpec((512, CW), lambda i: (0, 0)),
        ],
        out_specs=pl.BlockSpec((BM, CW), lambda i: (i, 0)),
        out_shape=jax.ShapeDtypeStruct((N, CW), jnp.float32),
    )(*([parts2] * (2 * C2)), g2, dinv, W2, b2, W3p)


def _k3_body(p0, p1, g3, dinv, b3, out_o):
    out_o[...] = (p0[0, 0] + p1[0, 0] + g3[...]) * dinv[...] + b3[...]


def _k3(parts3, g3, dinv, b3p):
    return pl.pallas_call(
        _k3_body,
        grid=(N // BM,),
        in_specs=[
            pl.BlockSpec((1, 1, BM, CW), lambda i: (0, 0, i, 0)),
            pl.BlockSpec((1, 1, BM, CW), lambda i: (1, 0, i, 0)),
            pl.BlockSpec((BM, CW), lambda i: (i, 0)),
            pl.BlockSpec((BM, 1), lambda i: (i, 0)),
            pl.BlockSpec((1, CW), lambda i: (0, 0)),
        ],
        out_specs=pl.BlockSpec((BM, CW), lambda i: (i, 0)),
        out_shape=jax.ShapeDtypeStruct((N, CW), jnp.float32),
    )(parts3, parts3, g3, dinv, b3p)


def _pad_tiles(a, padvals):
    t = a.reshape(NW, EPT)
    pad = jnp.broadcast_to(padvals, (NW, EPAD - EPT)).astype(jnp.int32)
    return jnp.concatenate([t, pad], axis=1).reshape(NC, NS, NB, BATCH)


def _table(g, C):
    flat = g.reshape(N * C, CW)
    return jnp.concatenate([flat, jnp.zeros((PADV * C, CW), jnp.float32)],
                           axis=0)


def kernel(x, edge_index, W1, b1, W2, b2, W3, b3):
    src = edge_index[0].astype(jnp.int32)
    dst = edge_index[1].astype(jnp.int32)
    k_pad = jnp.arange(EPAD - EPT, dtype=jnp.int32)
    src_p = _pad_tiles(src, N + (k_pad % PADV))
    dst_p = _pad_tiles(dst, N + (k_pad % (NPAD - N)))

    degp = _deg_kernel(dst_p)
    dinv, g1 = _k0(degp.reshape(NC, NPAD, 1), x)

    parts1 = _prop_l1(_table(g1, C1), src_p, dst_p)
    g2 = _k1(parts1, g1, dinv, W1, b1.reshape(1, 512))

    parts2 = _prop_l2(_table(g2, C2), src_p, dst_p)
    W3p = jnp.concatenate([W3, jnp.zeros((512, CW - 2), jnp.float32)], axis=1)
    g3 = _k2(parts2, g2, dinv, W2, b2.reshape(1, 512), W3p)

    parts3 = _prop_l3(_table(g3, 1), src_p, dst_p)
    b3p = jnp.concatenate([b3, jnp.zeros((CW - 2,), jnp.float32)]).reshape(1, CW)
    out128 = _k3(parts3, g3, dinv, b3p)
    return out128[:, :2]


# speedup vs baseline: 13.5337x; 1.0350x over previous
import functools

import jax
import jax.numpy as jnp
from jax import lax
from jax.experimental import pallas as pl
from jax.experimental.pallas import tpu as pltpu
from jax.experimental.pallas import tpu_sc as plsc

N = 10000
E = 160000
NC, NS, LANES = 2, 16, 16
NW = NC * NS
EPT = E // NW
BATCH = 128
NB = 40
EPAD = NB * BATCH
PADV = 16
NPAD = 10240
SPAN = NPAD // NS
BM = 400

_MESH = plsc.VectorSubcoreMesh(
    core_axis_name="c", subcore_axis_name="s", num_cores=NC, num_subcores=NS
)


def _zero_fill(buf, rows, cw):
    z16 = jnp.zeros((LANES,), jnp.float32)

    def body(r, _):
        for k in range(cw // LANES):
            buf[r, pl.ds(k * LANES, LANES)] = z16
        return 0

    lax.fori_loop(0, rows, body, 0)


CW = 128


def _make_propagate(C):
    @functools.partial(
        pl.kernel,
        out_type=jax.ShapeDtypeStruct((NC, C, NPAD, CW), jnp.float32),
        mesh=_MESH,
        scratch_types=[
            pltpu.VMEM((NB, BATCH), jnp.int32),
            pltpu.VMEM((NB, BATCH), jnp.int32),
            pltpu.VMEM((BATCH, CW), jnp.float32),
            pltpu.VMEM((BATCH, CW), jnp.float32),
            pltpu.VMEM_SHARED((NPAD, CW), jnp.float32),
            pltpu.SemaphoreType.DMA,
            pltpu.SemaphoreType.DMA,
            pltpu.SemaphoreType.DMA,
            pltpu.SemaphoreType.DMA,
        ],
    )
    def prop(table, src_p, dst_p, out, sidx_v, didx_v, rows0, rows1,
             acc, gsem0, gsem1, ssem0, ssem1):
        core = lax.axis_index("c")
        sid = lax.axis_index("s")
        pltpu.sync_copy(dst_p.at[core, sid], didx_v)
        pltpu.sync_copy(src_p.at[core, sid], sidx_v)
        if C > 1:
            def mulbody(r, _):
                for k in range(BATCH // LANES):
                    s = sidx_v[r, pl.ds(k * LANES, LANES)]
                    sidx_v[r, pl.ds(k * LANES, LANES)] = s * C
                return 0

            lax.fori_loop(0, NB, mulbody, 0)
        rows = (rows0, rows1)
        gsems = (gsem0, gsem1)
        ssems = (ssem0, ssem1)
        for c in range(C):
            _zero_fill(rows0, BATCH, CW)
            for z in range(SPAN // BATCH):
                pltpu.sync_copy(
                    rows0, acc.at[pl.ds(sid * SPAN + z * BATCH, BATCH)])
            if c > 0:
                def incbody(r, _):
                    for k in range(BATCH // LANES):
                        s = sidx_v[r, pl.ds(k * LANES, LANES)]
                        sidx_v[r, pl.ds(k * LANES, LANES)] = s + 1
                    return 0

                lax.fori_loop(0, NB, incbody, 0)
            plsc.subcore_barrier()
            gcp = [None, None]
            scp = [None, None]
            gcp[0] = pltpu.async_copy(table.at[sidx_v.at[0]], rows[0],
                                      gsems[0])
            for j in range(NB):
                b = j % 2
                o = 1 - b
                gcp[b].wait()
                scp[b] = pltpu.async_copy(rows[b], acc.at[didx_v.at[j]],
                                          ssems[b], add=True)
                if j + 1 < NB:
                    if scp[o] is not None:
                        scp[o].wait()
                        scp[o] = None
                    gcp[o] = pltpu.async_copy(
                        table.at[sidx_v.at[j + 1]], rows[o], gsems[o])
            for b in range(2):
                if scp[b] is not None:
                    scp[b].wait()
            plsc.subcore_barrier()
            pltpu.sync_copy(acc.at[pl.ds(sid * SPAN, SPAN)],
                            out.at[core, c, pl.ds(sid * SPAN, SPAN)])

    return prop


C1 = 2
C2 = 4
_prop_l1 = _make_propagate(C1)
_prop_l2 = _make_propagate(C2)
_prop_l3 = _make_propagate(1)


@functools.partial(
    pl.kernel,
    out_type=jax.ShapeDtypeStruct((NC, NPAD), jnp.float32),
    mesh=_MESH,
    scratch_types=[
        pltpu.VMEM((NB, BATCH), jnp.int32),
        pltpu.VMEM((BATCH,), jnp.float32),
        pltpu.VMEM((SPAN,), jnp.float32),
        pltpu.VMEM_SHARED((NPAD,), jnp.float32),
    ],
)
def _deg_kernel(dst_idx, out, didx_v, ones_v, zer_v, dacc):
    core = lax.axis_index("c")
    sid = lax.axis_index("s")
    one16 = jnp.ones((LANES,), jnp.float32)
    z16 = jnp.zeros((LANES,), jnp.float32)
    for k in range(BATCH // LANES):
        ones_v[pl.ds(k * LANES, LANES)] = one16
    def zbody(i, _):
        zer_v[pl.ds(i * LANES, LANES)] = z16
        return 0

    lax.fori_loop(0, SPAN // LANES, zbody, 0)
    pltpu.sync_copy(zer_v, dacc.at[pl.ds(sid * SPAN, SPAN)])
    plsc.subcore_barrier()
    pltpu.sync_copy(dst_idx.at[core, sid], didx_v)
    for j in range(NB):
        pltpu.sync_copy(ones_v, dacc.at[didx_v.at[j]], add=True)
    plsc.subcore_barrier()
    pltpu.sync_copy(dacc.at[pl.ds(sid * SPAN, SPAN)],
                    out.at[core, pl.ds(sid * SPAN, SPAN)])


def _k0_body(d0, d1, x, dinv_o, g1_o):
    deg = d0[0] + d1[0] + 1.0
    dv = lax.rsqrt(deg)
    dinv_o[...] = dv
    g1_o[...] = x[...] * dv


def _k0(degp, x):
    return pl.pallas_call(
        _k0_body,
        grid=(N // BM,),
        in_specs=[
            pl.BlockSpec((1, BM, 1), lambda i: (0, i, 0)),
            pl.BlockSpec((1, BM, 1), lambda i: (1, i, 0)),
            pl.BlockSpec((BM, 256), lambda i: (i, 0)),
        ],
        out_specs=[
            pl.BlockSpec((BM, 1), lambda i: (i, 0)),
            pl.BlockSpec((BM, 256), lambda i: (i, 0)),
        ],
        out_shape=[
            jax.ShapeDtypeStruct((N, 1), jnp.float32),
            jax.ShapeDtypeStruct((N, 256), jnp.float32),
        ],
    )(degp, degp, x)


def _part_specs(C):
    specs = []
    for c in range(C):
        for core in range(NC):
            specs.append(pl.BlockSpec(
                (1, 1, BM, CW),
                functools.partial(lambda i, _co=0, _ch=0: (_co, _ch, i, 0),
                                  _co=core, _ch=c)))
    return specs


def _psum(ps, C):
    return jnp.concatenate(
        [ps[2 * c][0, 0] + ps[2 * c + 1][0, 0] for c in range(C)], axis=1)


def _k1_body(*refs):
    ps = refs[:2 * C1]
    g1, dinv, w1, b1, g2_o = refs[2 * C1:]
    pre = _psum(ps, C1) + g1[...]
    dv = dinv[...]
    pre = pre * dv
    h = jnp.dot(pre.astype(jnp.bfloat16), w1[...].astype(jnp.bfloat16),
                preferred_element_type=jnp.float32) + b1[...]
    g2_o[...] = jnp.maximum(h, 0.0) * dv


def _k1(parts1, g1, dinv, W1, b1):
    return pl.pallas_call(
        _k1_body,
        grid=(N // BM,),
        in_specs=_part_specs(C1) + [
            pl.BlockSpec((BM, 256), lambda i: (i, 0)),
            pl.BlockSpec((BM, 1), lambda i: (i, 0)),
            pl.BlockSpec((256, 512), lambda i: (0, 0)),
            pl.BlockSpec((1, 512), lambda i: (0, 0)),
        ],
        out_specs=pl.BlockSpec((BM, 512), lambda i: (i, 0)),
        out_shape=jax.ShapeDtypeStruct((N, 512), jnp.float32),
    )(*([parts1] * (2 * C1)), g1, dinv, W1, b1)


def _k2_body(*refs):
    ps = refs[:2 * C2]
    g2, dinv, w2, b2, w3, g3_o = refs[2 * C2:]
    pre = _psum(ps, C2) + g2[...]
    dv = dinv[...]
    pre = pre * dv
    h = jnp.maximum(
        jnp.dot(pre.astype(jnp.bfloat16), w2[...].astype(jnp.bfloat16),
                preferred_element_type=jnp.float32) + b2[...], 0.0)
    g3_o[...] = jnp.dot(h.astype(jnp.bfloat16), w3[...].astype(jnp.bfloat16),
                        preferred_element_type=jnp.float32) * dv


def _k2(parts2, g2, dinv, W2, b2, W3p):
    return pl.pallas_call(
        _k2_body,
        grid=(N // BM,),
        in_specs=_part_specs(C2) + [
            pl.BlockSpec((BM, 512), lambda i: (i, 0)),
            pl.BlockSpec((BM, 1), lambda i: (i, 0)),
            pl.BlockSpec((512, 512), lambda i: (0, 0)),
            pl.BlockSpec((1, 512), lambda i: (0, 0)),
            pl.BlockSpec((512, CW), lambda i: (0, 0)),
        ],
        out_specs=pl.BlockSpec((BM, CW), lambda i: (i, 0)),
        out_shape=jax.ShapeDtypeStruct((N, CW), jnp.float32),
    )(*([parts2] * (2 * C2)), g2, dinv, W2, b2, W3p)


def _k3_body(p0, p1, g3, dinv, b3, out_o):
    out_o[...] = (p0[0, 0] + p1[0, 0] + g3[...]) * dinv[...] + b3[...]


def _k3(parts3, g3, dinv, b3p):
    return pl.pallas_call(
        _k3_body,
        grid=(N // BM,),
        in_specs=[
            pl.BlockSpec((1, 1, BM, CW), lambda i: (0, 0, i, 0)),
            pl.BlockSpec((1, 1, BM, CW), lambda i: (1, 0, i, 0)),
            pl.BlockSpec((BM, CW), lambda i: (i, 0)),
            pl.BlockSpec((BM, 1), lambda i: (i, 0)),
            pl.BlockSpec((1, CW), lambda i: (0, 0)),
        ],
        out_specs=pl.BlockSpec((BM, CW), lambda i: (i, 0)),
        out_shape=jax.ShapeDtypeStruct((N, CW), jnp.float32),
    )(parts3, parts3, g3, dinv, b3p)


def _pad_tiles(a, padvals):
    t = a.reshape(NW, EPT)
    pad = jnp.broadcast_to(padvals, (NW, EPAD - EPT)).astype(jnp.int32)
    return jnp.concatenate([t, pad], axis=1).reshape(NC, NS, NB, BATCH)


def _table(g, C):
    return g.reshape(N * C, CW)


def kernel(x, edge_index, W1, b1, W2, b2, W3, b3):
    src = edge_index[0].astype(jnp.int32)
    dst = edge_index[1].astype(jnp.int32)
    k_pad = jnp.arange(EPAD - EPT, dtype=jnp.int32)
    src_p = _pad_tiles(src, k_pad % PADV)
    dst_p = _pad_tiles(dst, N + (k_pad % (NPAD - N)))

    degp = _deg_kernel(dst_p)
    dinv, g1 = _k0(degp.reshape(NC, NPAD, 1), x)

    parts1 = _prop_l1(_table(g1, C1), src_p, dst_p)
    g2 = _k1(parts1, g1, dinv, W1, b1.reshape(1, 512))

    parts2 = _prop_l2(_table(g2, C2), src_p, dst_p)
    W3p = jnp.concatenate([W3, jnp.zeros((512, CW - 2), jnp.float32)], axis=1)
    g3 = _k2(parts2, g2, dinv, W2, b2.reshape(1, 512), W3p)

    parts3 = _prop_l3(_table(g3, 1), src_p, dst_p)
    b3p = jnp.concatenate([b3, jnp.zeros((CW - 2,), jnp.float32)]).reshape(1, CW)
    out128 = _k3(parts3, g3, dinv, b3p)
    return out128[:, :2]
